# Initial kernel scaffold; baseline (speedup 1.0000x reference)
import jax
import jax.numpy as jnp
from jax.experimental import pallas as pl


def kernel(indices, table):
    raise NotImplementedError("write your pallas kernel here")


# speedup vs baseline: 4.0873x; 4.0873x over previous
import functools

import jax
import jax.numpy as jnp
from jax import lax
from jax.experimental import pallas as pl
from jax.experimental.pallas import tpu as pltpu
from jax.experimental.pallas import tpu_sc as plsc

VOCAB = 100000
EMBED_DIM = 64
BATCH = 4096
HIST = 50

N = BATCH * HIST
NC, NS = 2, 16
NW = NC * NS
CHUNK = 128
NCHUNKS = N // CHUNK
CPW = NCHUNKS // NW


@functools.partial(
    pl.kernel,
    mesh=plsc.VectorSubcoreMesh(core_axis_name="c", subcore_axis_name="s"),
    compiler_params=pltpu.CompilerParams(use_tc_tiling_on_sc=False),
    out_type=jax.ShapeDtypeStruct((N, EMBED_DIM), jnp.float32),
    scratch_types=[
        pltpu.VMEM((CPW * CHUNK,), jnp.int32),
        pltpu.VMEM((CHUNK, EMBED_DIM), jnp.float32),
        pltpu.SemaphoreType.DMA,
    ],
)
def _emb_gather(idx_hbm, table_hbm, out_hbm, idx_v, rows_v, sem):
    wid = lax.axis_index("s") * NC + lax.axis_index("c")
    base = wid * CPW * CHUNK
    pltpu.sync_copy(idx_hbm.at[pl.ds(base, CPW * CHUNK)], idx_v)

    def body(j, carry):
        idx_chunk = idx_v.at[pl.ds(j * CHUNK, CHUNK)]
        pltpu.async_copy(table_hbm.at[idx_chunk], rows_v, sem).wait()
        pltpu.sync_copy(rows_v, out_hbm.at[pl.ds(base + j * CHUNK, CHUNK)])
        return carry

    lax.fori_loop(0, CPW, body, 0)


def kernel(indices, table):
    idx2 = indices.reshape(N).astype(jnp.int32)
    out = _emb_gather(idx2, table)
    return out.reshape(BATCH, HIST, EMBED_DIM)


# speedup vs baseline: 4.6380x; 1.1347x over previous
import functools

import jax
import jax.numpy as jnp
from jax import lax
from jax.experimental import pallas as pl
from jax.experimental.pallas import tpu as pltpu
from jax.experimental.pallas import tpu_sc as plsc

VOCAB = 100000
EMBED_DIM = 64
BATCH = 4096
HIST = 50

N = BATCH * HIST
NC, NS = 2, 16
NW = NC * NS
CHUNK = 128
NCHUNKS = N // CHUNK
CPW = NCHUNKS // NW
GROUP = 5
GROUP_ROWS = GROUP * CHUNK
NGROUP = CPW // GROUP


@functools.partial(
    pl.kernel,
    mesh=plsc.VectorSubcoreMesh(core_axis_name="c", subcore_axis_name="s"),
    compiler_params=pltpu.CompilerParams(use_tc_tiling_on_sc=False),
    out_type=jax.ShapeDtypeStruct((N, EMBED_DIM), jnp.float32),
    scratch_types=[
        pltpu.VMEM((CPW * CHUNK,), jnp.int32),
        pltpu.VMEM((GROUP_ROWS, EMBED_DIM), jnp.float32),
        pltpu.VMEM((GROUP_ROWS, EMBED_DIM), jnp.float32),
        pltpu.SemaphoreType.DMA,
        pltpu.SemaphoreType.DMA,
        pltpu.SemaphoreType.DMA,
        pltpu.SemaphoreType.DMA,
    ],
)
def _emb_gather(idx_hbm, table_hbm, out_hbm, idx_v, rows_a, rows_b,
                gsem_a, gsem_b, osem_a, osem_b):
    wid = lax.axis_index("s") * NC + lax.axis_index("c")
    base = wid * CPW * CHUNK
    pltpu.sync_copy(idx_hbm.at[pl.ds(base, CPW * CHUNK)], idx_v)

    rows = (rows_a, rows_b)
    gsem = (gsem_a, gsem_b)
    osem = (osem_a, osem_b)

    def fire_gathers(g):
        s = g % 2
        handles = []
        for b in range(GROUP):
            j = g * GROUP + b
            idx_chunk = idx_v.at[pl.ds(j * CHUNK, CHUNK)]
            dst = rows[s].at[pl.ds(b * CHUNK, CHUNK)]
            handles.append(pltpu.async_copy(table_hbm.at[idx_chunk], dst, gsem[s]))
        return handles

    def fire_outcopy(g):
        s = g % 2
        dst = out_hbm.at[pl.ds(base + g * GROUP_ROWS, GROUP_ROWS)]
        return pltpu.async_copy(rows[s], dst, osem[s])

    gathers = {0: fire_gathers(0)}
    outcopies = {}
    for g in range(NGROUP):
        if g >= 1:
            outcopies.pop(g - 1).wait()
        if g + 1 < NGROUP:
            gathers[g + 1] = fire_gathers(g + 1)
        for h in gathers.pop(g):
            h.wait()
        outcopies[g] = fire_outcopy(g)
    outcopies.pop(NGROUP - 1).wait()


def kernel(indices, table):
    idx2 = indices.reshape(N).astype(jnp.int32)
    out = _emb_gather(idx2, table)
    return out.reshape(BATCH, HIST, EMBED_DIM)
